# baseline (speedup 1.0000x reference)
import functools

import numpy as np
import jax
import jax.numpy as jnp
from jax import lax
from jax.experimental import pallas as pl
from jax.experimental.pallas import tpu as pltpu
from jax.experimental.pallas import tpu_sc as plsc

_LENGTH = 200
_D = 64
_BATCH = 4096
_N = _BATCH * _LENGTH
_NC = 2
_NS = 16
_NW = _NC * _NS
_K = 128
_CHUNKS = _N // (_NW * _K)
_SCALE = 8.0


def _pos_enc_doubled() -> np.ndarray:
    depth = _D / 2
    positions = np.arange(_LENGTH)[:, None]
    depths = np.arange(depth)[None, :] / depth
    angle_rates = 1 / 10000 ** depths
    angle_rads = positions * angle_rates
    pe = np.concatenate([np.sin(angle_rads), np.cos(angle_rads)], axis=-1)
    pe = pe.astype(np.float32)
    return np.concatenate([pe, pe], axis=0).reshape(-1)


_mesh = plsc.VectorSubcoreMesh(core_axis_name="c", subcore_axis_name="s")


@functools.partial(
    pl.kernel,
    mesh=_mesh,
    out_type=jax.ShapeDtypeStruct((_N, _D), jnp.float32),
    scratch_types=[
        pltpu.VMEM((_CHUNKS, _K), jnp.int32),
        pltpu.VMEM((2 * _LENGTH * _D,), jnp.float32),
        pltpu.VMEM((_K, _D), jnp.float32),
        pltpu.SemaphoreType.DMA,
    ],
    compiler_params=pltpu.CompilerParams(use_tc_tiling_on_sc=False),
)
def _sc_embed(x_hbm, table_hbm, pe_hbm, out_hbm, idx_v, pe_v, rows_v, sem):
    wid = lax.axis_index("s") * _NC + lax.axis_index("c")
    pltpu.sync_copy(x_hbm.at[pl.ds(wid * _CHUNKS, _CHUNKS)], idx_v)
    pltpu.sync_copy(pe_hbm, pe_v)

    def chunk_body(c, _):
        pltpu.async_copy(table_hbm.at[idx_v.at[c]], rows_v, sem).wait()
        s = lax.rem(c * _K, _LENGTH)

        def row_body(r, _):
            pe_base = (s + r) * _D
            for q in range(_D // 16):
                v = rows_v[r, pl.ds(q * 16, 16)]
                p = pe_v[pl.ds(pe_base + q * 16, 16)]
                rows_v[r, pl.ds(q * 16, 16)] = v * _SCALE + p
            return 0

        lax.fori_loop(0, _K, row_body, 0)
        g = wid * (_CHUNKS * _K) + c * _K
        pltpu.sync_copy(rows_v, out_hbm.at[pl.ds(g, _K)])
        return 0

    lax.fori_loop(0, _CHUNKS, chunk_body, 0)


def kernel(x, embedding_table):
    x2d = x.reshape(_NW * _CHUNKS, _K).astype(jnp.int32)
    pe = jnp.asarray(_pos_enc_doubled())
    out = _sc_embed(x2d, embedding_table, pe)
    return out.reshape(_BATCH, _LENGTH, _D)


# speedup vs baseline: 1.7422x; 1.7422x over previous
import functools

import numpy as np
import jax
import jax.numpy as jnp
from jax import lax
from jax.experimental import pallas as pl
from jax.experimental.pallas import tpu as pltpu
from jax.experimental.pallas import tpu_sc as plsc

_LENGTH = 200
_D = 64
_BATCH = 4096
_NC = 2
_NS = 16
_NW = _NC * _NS
_BW = _BATCH // _NW
_NB = 4
_AHEAD = 2
_SCALE = 8.0


def _pos_enc() -> np.ndarray:
    depth = _D / 2
    positions = np.arange(_LENGTH)[:, None]
    depths = np.arange(depth)[None, :] / depth
    angle_rates = 1 / 10000 ** depths
    angle_rads = positions * angle_rates
    pe = np.concatenate([np.sin(angle_rads), np.cos(angle_rads)], axis=-1)
    return pe.astype(np.float32)


_mesh = plsc.VectorSubcoreMesh(core_axis_name="c", subcore_axis_name="s")


@functools.partial(
    pl.kernel,
    mesh=_mesh,
    out_type=jax.ShapeDtypeStruct((_BATCH, _LENGTH * _D), jnp.float32),
    scratch_types=[
        pltpu.VMEM((_LENGTH, _BW), jnp.int32),
        pltpu.VMEM((_LENGTH, _D), jnp.float32),
        pltpu.VMEM((_NB, _BW, _D), jnp.float32),
    ] + [pltpu.SemaphoreType.DMA] * (2 * _NB),
    compiler_params=pltpu.CompilerParams(use_tc_tiling_on_sc=False),
)
def _sc_embed(xt_hbm, table_hbm, pe_hbm, out_hbm, idx_v, pe_v, rows_v, *sems):
    gsem, ssem = sems[:_NB], sems[_NB:]
    wid = lax.axis_index("s") * _NC + lax.axis_index("c")
    b0 = wid * _BW
    pltpu.sync_copy(xt_hbm.at[:, pl.ds(b0, _BW)], idx_v)
    pltpu.sync_copy(pe_hbm, pe_v)

    def gather(t, b):
        t = lax.min(t, _LENGTH - 1)
        return pltpu.make_async_copy(
            table_hbm.at[idx_v.at[t]], rows_v.at[b], gsem[b])

    def scat(t, b):
        t = lax.max(t, 0)
        return pltpu.make_async_copy(
            rows_v.at[b], out_hbm.at[pl.ds(b0, _BW), pl.ds(t * _D, _D)],
            ssem[b])

    for b in range(_AHEAD):
        gather(b, b).start()

    def substep(t, b):
        bn = (b + _AHEAD) % _NB
        @pl.when(t >= _NB - _AHEAD)
        def _():
            scat(t - (_NB - _AHEAD), bn).wait()

        @pl.when(t + _AHEAD < _LENGTH)
        def _():
            gather(t + _AHEAD, bn).start()

        gather(t, b).wait()
        pq = tuple(pe_v[t, pl.ds(16 * q, 16)] for q in range(_D // 16))

        def row_body(r, carry):
            for q in range(_D // 16):
                v = rows_v[b, r, pl.ds(16 * q, 16)]
                rows_v[b, r, pl.ds(16 * q, 16)] = v * _SCALE + carry[q]
            return carry

        lax.fori_loop(0, _BW, row_body, pq, unroll=4)
        scat(t, b).start()

    def outer(i, _):
        for b in range(_NB):
            substep(i * _NB + b, b)
        return 0

    lax.fori_loop(0, _LENGTH // _NB, outer, 0)
    for k in range(_NB - _AHEAD, _NB):
        scat(_LENGTH - _NB + k, k).wait()


def kernel(x, embedding_table):
    xt = x.T.astype(jnp.int32)
    pe = jnp.asarray(_pos_enc())
    out = _sc_embed(xt, embedding_table, pe)
    return out.reshape(_BATCH, _LENGTH, _D)
